# baseline (speedup 1.0000x reference)
import jax
import jax.numpy as jnp
from jax import lax
from jax.experimental import pallas as pl
from jax.experimental.pallas import tpu as pltpu
from jax.experimental.pallas import tpu_sc as plsc

_NC = 2
_NS = 16
_NW = _NC * _NS
_L = 16
_D = 32
_V = 1000000
_SUB = 8
_LANES = 128
_TCOLS = -(-_V // _LANES)
_SEG = _TCOLS * _SUB * _LANES


def _mf_body(uidx_hbm, iidx_hbm, uflat_hbm, iflat_hbm, out_hbm,
             uidx_v, iidx_v, urT_v, irT_v, out_v,
             sem_u, sem_i):
    b_per_w = uidx_v.shape[0]
    wid = lax.axis_index("s") * _NC + lax.axis_index("c")
    base = wid * b_per_w

    pltpu.sync_copy(uidx_hbm.at[pl.ds(base, b_per_w)], uidx_v)
    pltpu.sync_copy(iidx_hbm.at[pl.ds(base, b_per_w)], iidx_v)

    copies = []
    for j in range(_D):
        copies.append(pltpu.async_copy(
            uflat_hbm.at[j].at[uidx_v],
            urT_v.at[pl.ds(j * b_per_w, b_per_w)], sem_u))
        copies.append(pltpu.async_copy(
            iflat_hbm.at[j].at[iidx_v],
            irT_v.at[pl.ds(j * b_per_w, b_per_w)], sem_i))
    for cp in copies:
        cp.wait()

    def chunk_body(c, carry):
        b0 = c * _L
        acc = jnp.zeros((_L,), jnp.float32)
        for j in range(_D):
            u = urT_v[pl.ds(j * b_per_w + b0, _L)]
            i = irT_v[pl.ds(j * b_per_w + b0, _L)]
            acc = acc + u * i
        out_v[pl.ds(b0, _L)] = 1.0 / (1.0 + jnp.exp(-acc))
        return carry

    lax.fori_loop(0, b_per_w // _L, chunk_body, 0)
    pltpu.sync_copy(out_v, out_hbm.at[pl.ds(base, b_per_w)])


def _dim_major(table):
    return table.T


def kernel(user_indices, item_indices, user_table, item_table):
    B = user_indices.shape[0]
    assert B % (_NW * _L) == 0
    assert user_table.shape == (_V, _D)
    b_per_w = B // _NW
    mesh = plsc.VectorSubcoreMesh(core_axis_name="c", subcore_axis_name="s",
                                  num_cores=_NC, num_subcores=_NS)
    run = pl.kernel(
        _mf_body,
        out_type=jax.ShapeDtypeStruct((B,), jnp.float32),
        mesh=mesh,
        compiler_params=pltpu.CompilerParams(needs_layout_passes=False,
                                             use_tc_tiling_on_sc=False),
        scratch_types=[
            pltpu.VMEM((b_per_w,), jnp.int32),
            pltpu.VMEM((b_per_w,), jnp.int32),
            pltpu.VMEM((_D * b_per_w,), jnp.float32),
            pltpu.VMEM((_D * b_per_w,), jnp.float32),
            pltpu.VMEM((b_per_w,), jnp.float32),
            pltpu.SemaphoreType.DMA,
            pltpu.SemaphoreType.DMA,
        ],
    )
    return run(user_indices, item_indices,
               _dim_major(user_table), _dim_major(item_table))


# speedup vs baseline: 4.8772x; 4.8772x over previous
import jax
import jax.numpy as jnp
from jax import lax
from jax.experimental import pallas as pl
from jax.experimental.pallas import tpu as pltpu
from jax.experimental.pallas import tpu_sc as plsc

_NC = 2
_NS = 16
_NW = _NC * _NS
_L = 16
_D = 32


def _mf_body(uidx_hbm, iidx_hbm, utab_hbm, itab_hbm, out_hbm,
             uidx_v, iidx_v, urows_v, irows_v, q_v, out_v, sem_u, sem_i):
    b_per_w = uidx_v.shape[0]
    wid = lax.axis_index("s") * _NC + lax.axis_index("c")
    base = wid * b_per_w

    pltpu.sync_copy(uidx_hbm.at[pl.ds(base, b_per_w)], uidx_v)
    pltpu.sync_copy(iidx_hbm.at[pl.ds(base, b_per_w)], iidx_v)
    cp_u = pltpu.async_copy(utab_hbm.at[uidx_v], urows_v, sem_u)
    cp_i = pltpu.async_copy(itab_hbm.at[iidx_v], irows_v, sem_i)
    cp_u.wait()
    cp_i.wait()

    lanes = lax.iota(jnp.int32, _L)

    def chunk_body(c, carry):
        b0 = c * _L
        for k in range(_L):
            u0, u1 = plsc.unpack(urows_v[b0 + k, 0:32],
                                 format=plsc.PackFormat.INTERLEAVED)
            i0, i1 = plsc.unpack(irows_v[b0 + k, 0:32],
                                 format=plsc.PackFormat.INTERLEAVED)
            q_v[pl.ds(k * (_L + 1), _L)] = u0 * i0 + u1 * i1
        acc = jnp.zeros((_L,), jnp.float32)
        stride_lanes = lanes * (_L + 1)
        for j in range(_L):
            acc = acc + plsc.load_gather(q_v, [stride_lanes + j])
        out_v[pl.ds(b0, _L)] = 1.0 / (1.0 + jnp.exp(-acc))
        return carry

    lax.fori_loop(0, b_per_w // _L, chunk_body, 0)
    pltpu.sync_copy(out_v, out_hbm.at[pl.ds(base, b_per_w)])


def kernel(user_indices, item_indices, user_table, item_table):
    B = user_indices.shape[0]
    assert B % (_NW * _L) == 0
    b_per_w = B // _NW
    mesh = plsc.VectorSubcoreMesh(core_axis_name="c", subcore_axis_name="s",
                                  num_cores=_NC, num_subcores=_NS)
    run = pl.kernel(
        _mf_body,
        out_type=jax.ShapeDtypeStruct((B,), jnp.float32),
        mesh=mesh,
        compiler_params=pltpu.CompilerParams(needs_layout_passes=False,
                                             use_tc_tiling_on_sc=False),
        scratch_types=[
            pltpu.VMEM((b_per_w,), jnp.int32),
            pltpu.VMEM((b_per_w,), jnp.int32),
            pltpu.VMEM((b_per_w, _D), jnp.bfloat16),
            pltpu.VMEM((b_per_w, _D), jnp.bfloat16),
            pltpu.VMEM((_L * (_L + 1),), jnp.float32),
            pltpu.VMEM((b_per_w,), jnp.float32),
            pltpu.SemaphoreType.DMA,
            pltpu.SemaphoreType.DMA,
        ],
    )
    return run(user_indices, item_indices,
               user_table.astype(jnp.bfloat16),
               item_table.astype(jnp.bfloat16))


# speedup vs baseline: 18.9663x; 3.8887x over previous
import jax
import jax.numpy as jnp
from jax import lax
from jax.experimental import pallas as pl
from jax.experimental.pallas import tpu as pltpu
from jax.experimental.pallas import tpu_sc as plsc

_NC = 2
_NS = 16
_NW = _NC * _NS
_L = 16
_D = 32
_G = 8


def _mf_body(uidx_hbm, iidx_hbm, utabT_hbm, itabT_hbm, out_hbm,
             uidx_v, iidx_v, slab_u, slab_i, q_v, out_v,
             sem_u, sem_i):
    b_per_w = uidx_v.shape[0]
    wid = lax.axis_index("s") * _NC + lax.axis_index("c")
    base = wid * b_per_w

    pltpu.sync_copy(uidx_hbm.at[pl.ds(base, b_per_w)], uidx_v)
    pltpu.sync_copy(iidx_hbm.at[pl.ds(base, b_per_w)], iidx_v)

    lanes = lax.iota(jnp.int32, _L)

    def super_body(c, carry):
        b0 = c * _L
        u16 = uidx_v[pl.ds(b0, _L)]
        i16 = iidx_v[pl.ds(b0, _L)]
        utile = u16 >> 7
        itile = i16 >> 7
        ucol = u16 & 127
        icol = i16 & 127
        for sub in range(_L // _G):
            copies = []
            for k in range(_G):
                ka = sub * _G + k
                su = jnp.sum(jnp.where(lanes == ka, utile, 0))
                si = jnp.sum(jnp.where(lanes == ka, itile, 0))
                cu0 = pl.multiple_of(su * 128, 128)
                ci0 = pl.multiple_of(si * 128, 128)
                copies.append(pltpu.async_copy(
                    utabT_hbm.at[:, pl.ds(cu0, 128)],
                    slab_u.at[pl.ds(k * _D, _D), :], sem_u))
                copies.append(pltpu.async_copy(
                    itabT_hbm.at[:, pl.ds(ci0, 128)],
                    slab_i.at[pl.ds(k * _D, _D), :], sem_i))
            for cp in copies:
                cp.wait()
            for k in range(_G):
                ka = sub * _G + k
                kvec = jnp.full((_L,), ka, jnp.int32)
                cu = jnp.take_along_axis(ucol, kvec, axis=0)
                ci = jnp.take_along_axis(icol, kvec, axis=0)
                u0 = plsc.load_gather(slab_u, [k * _D + lanes, cu])
                u1 = plsc.load_gather(slab_u, [k * _D + _L + lanes, cu])
                i0 = plsc.load_gather(slab_i, [k * _D + lanes, ci])
                i1 = plsc.load_gather(slab_i, [k * _D + _L + lanes, ci])
                q_v[pl.ds(ka * (_L + 1), _L)] = u0 * i0 + u1 * i1
        acc = jnp.zeros((_L,), jnp.float32)
        stride_lanes = lanes * (_L + 1)
        for j in range(_L):
            acc = acc + plsc.load_gather(q_v, [stride_lanes + j])
        out_v[pl.ds(b0, _L)] = 1.0 / (1.0 + jnp.exp(-acc))
        return carry

    lax.fori_loop(0, b_per_w // _L, super_body, 0)
    pltpu.sync_copy(out_v, out_hbm.at[pl.ds(base, b_per_w)])


def kernel(user_indices, item_indices, user_table, item_table):
    B = user_indices.shape[0]
    assert B % (_NW * _L) == 0
    b_per_w = B // _NW
    mesh = plsc.VectorSubcoreMesh(core_axis_name="c", subcore_axis_name="s",
                                  num_cores=_NC, num_subcores=_NS)
    run = pl.kernel(
        _mf_body,
        out_type=jax.ShapeDtypeStruct((B,), jnp.float32),
        mesh=mesh,
        compiler_params=pltpu.CompilerParams(needs_layout_passes=False),
        scratch_types=[
            pltpu.VMEM((b_per_w,), jnp.int32),
            pltpu.VMEM((b_per_w,), jnp.int32),
            pltpu.VMEM((_G * _D, 128), jnp.float32),
            pltpu.VMEM((_G * _D, 128), jnp.float32),
            pltpu.VMEM((_L * (_L + 1),), jnp.float32),
            pltpu.VMEM((b_per_w,), jnp.float32),
            pltpu.SemaphoreType.DMA,
            pltpu.SemaphoreType.DMA,
        ],
    )
    return run(user_indices, item_indices, user_table.T, item_table.T)


# speedup vs baseline: 19.6078x; 1.0338x over previous
import jax
import jax.numpy as jnp
from jax import lax
from jax.experimental import pallas as pl
from jax.experimental.pallas import tpu as pltpu
from jax.experimental.pallas import tpu_sc as plsc

_NC = 2
_NS = 16
_NW = _NC * _NS
_L = 16
_D = 32
_G = 4


def _mf_body(uidx_hbm, iidx_hbm, utabT_hbm, itabT_hbm, out_hbm,
             uidx_v, iidx_v, slab_u, slab_i, q_v, out_v,
             sem_u0, sem_u1, sem_i0, sem_i1):
    b_per_w = uidx_v.shape[0]
    wid = lax.axis_index("s") * _NC + lax.axis_index("c")
    base = wid * b_per_w

    pltpu.sync_copy(uidx_hbm.at[pl.ds(base, b_per_w)], uidx_v)
    pltpu.sync_copy(iidx_hbm.at[pl.ds(base, b_per_w)], iidx_v)

    lanes = lax.iota(jnp.int32, _L)
    sems_u = (sem_u0, sem_u1)
    sems_i = (sem_i0, sem_i1)

    def super_body(c, carry):
        b0 = c * _L
        u16 = uidx_v[pl.ds(b0, _L)]
        i16 = iidx_v[pl.ds(b0, _L)]
        utile = u16 >> 7
        itile = i16 >> 7
        ucol = u16 & 127
        icol = i16 & 127

        def fire(sub):
            p = sub & 1
            copies = []
            for k in range(_G):
                ka = sub * _G + k
                su = jnp.sum(jnp.where(lanes == ka, utile, 0))
                si = jnp.sum(jnp.where(lanes == ka, itile, 0))
                cu0 = pl.multiple_of(su * 128, 128)
                ci0 = pl.multiple_of(si * 128, 128)
                row0 = (p * _G + k) * _D
                copies.append(pltpu.async_copy(
                    utabT_hbm.at[:, pl.ds(cu0, 128)],
                    slab_u.at[pl.ds(row0, _D), :], sems_u[p]))
                copies.append(pltpu.async_copy(
                    itabT_hbm.at[:, pl.ds(ci0, 128)],
                    slab_i.at[pl.ds(row0, _D), :], sems_i[p]))
            return copies

        def extract(sub):
            p = sub & 1
            for k in range(_G):
                ka = sub * _G + k
                kvec = jnp.full((_L,), ka, jnp.int32)
                cu = jnp.take_along_axis(ucol, kvec, axis=0)
                ci = jnp.take_along_axis(icol, kvec, axis=0)
                row0 = (p * _G + k) * _D
                u0 = plsc.load_gather(slab_u, [row0 + lanes, cu])
                u1 = plsc.load_gather(slab_u, [row0 + _L + lanes, cu])
                i0 = plsc.load_gather(slab_i, [row0 + lanes, ci])
                i1 = plsc.load_gather(slab_i, [row0 + _L + lanes, ci])
                q_v[pl.ds(ka * (_L + 1), _L)] = u0 * i0 + u1 * i1

        pending = fire(0)
        nxt = fire(1)
        for sub in range(_L // _G):
            for cp in pending:
                cp.wait()
            extract(sub)
            pending = nxt
            nxt = fire(sub + 2) if sub + 2 < _L // _G else []
        acc = jnp.zeros((_L,), jnp.float32)
        stride_lanes = lanes * (_L + 1)
        for j in range(_L):
            acc = acc + plsc.load_gather(q_v, [stride_lanes + j])
        out_v[pl.ds(b0, _L)] = 1.0 / (1.0 + jnp.exp(-acc))
        return carry

    lax.fori_loop(0, b_per_w // _L, super_body, 0)
    pltpu.sync_copy(out_v, out_hbm.at[pl.ds(base, b_per_w)])


def kernel(user_indices, item_indices, user_table, item_table):
    B = user_indices.shape[0]
    assert B % (_NW * _L) == 0
    assert user_table.shape[1] == _D
    b_per_w = B // _NW
    mesh = plsc.VectorSubcoreMesh(core_axis_name="c", subcore_axis_name="s",
                                  num_cores=_NC, num_subcores=_NS)
    run = pl.kernel(
        _mf_body,
        out_type=jax.ShapeDtypeStruct((B,), jnp.float32),
        mesh=mesh,
        compiler_params=pltpu.CompilerParams(needs_layout_passes=False),
        scratch_types=[
            pltpu.VMEM((b_per_w,), jnp.int32),
            pltpu.VMEM((b_per_w,), jnp.int32),
            pltpu.VMEM((2 * _G * _D, 128), jnp.float32),
            pltpu.VMEM((2 * _G * _D, 128), jnp.float32),
            pltpu.VMEM((_L * (_L + 1),), jnp.float32),
            pltpu.VMEM((b_per_w,), jnp.float32),
            pltpu.SemaphoreType.DMA,
            pltpu.SemaphoreType.DMA,
            pltpu.SemaphoreType.DMA,
            pltpu.SemaphoreType.DMA,
        ],
    )
    return run(user_indices, item_indices, user_table.T, item_table.T)


# speedup vs baseline: 19.6453x; 1.0019x over previous
import jax
import jax.numpy as jnp
from jax import lax
from jax.experimental import pallas as pl
from jax.experimental.pallas import tpu as pltpu
from jax.experimental.pallas import tpu_sc as plsc

_NC = 2
_NS = 16
_NW = _NC * _NS
_L = 16
_D = 32
_G = 4


def _mf_body(uidx_hbm, iidx_hbm, utabT_hbm, itabT_hbm, out_hbm,
             uidx_v, iidx_v, slab_u, slab_i, q_v, out_v,
             sem_u0, sem_u1, sem_u2, sem_i0, sem_i1, sem_i2):
    b_per_w = uidx_v.shape[0]
    wid = lax.axis_index("s") * _NC + lax.axis_index("c")
    base = wid * b_per_w

    pltpu.sync_copy(uidx_hbm.at[pl.ds(base, b_per_w)], uidx_v)
    pltpu.sync_copy(iidx_hbm.at[pl.ds(base, b_per_w)], iidx_v)

    lanes = lax.iota(jnp.int32, _L)
    sems_u = (sem_u0, sem_u1, sem_u2)
    sems_i = (sem_i0, sem_i1, sem_i2)

    def super_body(c, carry):
        b0 = c * _L
        u16 = uidx_v[pl.ds(b0, _L)]
        i16 = iidx_v[pl.ds(b0, _L)]
        utile = u16 >> 7
        itile = i16 >> 7
        ucol = u16 & 127
        icol = i16 & 127

        def fire(sub):
            p = sub % 3
            copies = []
            for k in range(_G):
                ka = sub * _G + k
                su = jnp.sum(jnp.where(lanes == ka, utile, 0))
                si = jnp.sum(jnp.where(lanes == ka, itile, 0))
                cu0 = pl.multiple_of(su * 128, 128)
                ci0 = pl.multiple_of(si * 128, 128)
                row0 = (p * _G + k) * _D
                copies.append(pltpu.async_copy(
                    utabT_hbm.at[:, pl.ds(cu0, 128)],
                    slab_u.at[pl.ds(row0, _D), :], sems_u[p]))
                copies.append(pltpu.async_copy(
                    itabT_hbm.at[:, pl.ds(ci0, 128)],
                    slab_i.at[pl.ds(row0, _D), :], sems_i[p]))
            return copies

        def extract(sub):
            p = sub % 3
            for k in range(_G):
                ka = sub * _G + k
                kvec = jnp.full((_L,), ka, jnp.int32)
                cu = jnp.take_along_axis(ucol, kvec, axis=0)
                ci = jnp.take_along_axis(icol, kvec, axis=0)
                row0 = (p * _G + k) * _D
                u0 = plsc.load_gather(slab_u, [row0 + lanes, cu])
                u1 = plsc.load_gather(slab_u, [row0 + _L + lanes, cu])
                i0 = plsc.load_gather(slab_i, [row0 + lanes, ci])
                i1 = plsc.load_gather(slab_i, [row0 + _L + lanes, ci])
                q_v[pl.ds(ka * (_L + 1), _L)] = u0 * i0 + u1 * i1

        inflight = [fire(0), fire(1), fire(2)]
        for sub in range(_L // _G):
            for cp in inflight.pop(0):
                cp.wait()
            extract(sub)
            if sub + 3 < _L // _G:
                inflight.append(fire(sub + 3))
            else:
                inflight.append([])
        acc = jnp.zeros((_L,), jnp.float32)
        stride_lanes = lanes * (_L + 1)
        for j in range(_L):
            acc = acc + plsc.load_gather(q_v, [stride_lanes + j])
        out_v[pl.ds(b0, _L)] = 1.0 / (1.0 + jnp.exp(-acc))
        return carry

    lax.fori_loop(0, b_per_w // _L, super_body, 0)
    pltpu.sync_copy(out_v, out_hbm.at[pl.ds(base, b_per_w)])


def kernel(user_indices, item_indices, user_table, item_table):
    B = user_indices.shape[0]
    assert B % (_NW * _L) == 0
    assert user_table.shape[1] == _D
    b_per_w = B // _NW
    mesh = plsc.VectorSubcoreMesh(core_axis_name="c", subcore_axis_name="s",
                                  num_cores=_NC, num_subcores=_NS)
    run = pl.kernel(
        _mf_body,
        out_type=jax.ShapeDtypeStruct((B,), jnp.float32),
        mesh=mesh,
        compiler_params=pltpu.CompilerParams(needs_layout_passes=False),
        scratch_types=[
            pltpu.VMEM((b_per_w,), jnp.int32),
            pltpu.VMEM((b_per_w,), jnp.int32),
            pltpu.VMEM((3 * _G * _D, 128), jnp.float32),
            pltpu.VMEM((3 * _G * _D, 128), jnp.float32),
            pltpu.VMEM((_L * (_L + 1),), jnp.float32),
            pltpu.VMEM((b_per_w,), jnp.float32),
            pltpu.SemaphoreType.DMA,
            pltpu.SemaphoreType.DMA,
            pltpu.SemaphoreType.DMA,
            pltpu.SemaphoreType.DMA,
            pltpu.SemaphoreType.DMA,
            pltpu.SemaphoreType.DMA,
        ],
    )
    return run(user_indices, item_indices, user_table.T, item_table.T)
